# Initial kernel scaffold; baseline (speedup 1.0000x reference)
import jax
import jax.numpy as jnp
from jax.experimental import pallas as pl


def kernel(pred, target):
    raise NotImplementedError("write your pallas kernel here")


# speedup vs baseline: 12.9128x; 12.9128x over previous
import functools

import jax
import jax.numpy as jnp
from jax.experimental import pallas as pl
from jax.experimental.pallas import tpu as pltpu

_HARD_RATIO = 0.3
_MIN_KEPT = 1000
_FOCAL_ALPHA = 0.25
_FOCAL_GAMMA = 2.0
_DICE_WEIGHT = 0.5
_FOCAL_WEIGHT = 0.5

_B = 16
_NPIX = 512 * 512
_K = min(max(int(_NPIX * _HARD_RATIO), _MIN_KEPT), _NPIX)


def _tc_body(pred_ref, target_ref, topk_ref, inter_ref, ssig_ref, st_ref):
    x = pred_ref[0]
    t = target_ref[0].astype(jnp.float32)

    bce = jnp.maximum(x, 0.0) - x * t + jnp.log1p(jnp.exp(-jnp.abs(x)))
    p_t = jnp.exp(-bce)
    focal = _FOCAL_ALPHA * (1.0 - p_t) ** 2 * bce

    sig = 1.0 / (1.0 + jnp.exp(-x))
    inter_ref[0, 0, 0] = jnp.sum(sig * t)
    ssig_ref[0, 0, 0] = jnp.sum(sig)
    st_ref[0, 0, 0] = jnp.sum(t)

    bits = jax.lax.bitcast_convert_type(focal, jnp.int32)

    def body(_, lohi):
        lo, hi = lohi
        d = hi - lo
        mid = lo + (d >> 1) + (d & 1)
        cnt = jnp.sum((bits >= mid).astype(jnp.int32))
        ok = cnt >= _K
        return jnp.where(ok, mid, lo), jnp.where(ok, hi, mid - 1)

    lo, _ = jax.lax.fori_loop(0, 31, body, (jnp.int32(0), jnp.int32(0x7FFFFFFF)))

    gt = bits > lo
    cnt_gt = jnp.sum(gt.astype(jnp.int32))
    sum_gt = jnp.sum(jnp.where(gt, focal, 0.0))
    tval = jax.lax.bitcast_convert_type(lo, jnp.float32)
    topk_ref[0, 0, 0] = sum_gt + (_K - cnt_gt).astype(jnp.float32) * tval


@jax.jit
def kernel(pred, target):
    pred2 = pred.reshape(_B, 2048, 128)
    target2 = target.reshape(_B, 2048, 128)

    out_shapes = [jax.ShapeDtypeStruct((_B, 1, 1), jnp.float32)] * 4
    scalar_spec = pl.BlockSpec((1, 1, 1), lambda i: (i, 0, 0),
                               memory_space=pltpu.SMEM)
    topk, inter, ssig, st = pl.pallas_call(
        _tc_body,
        grid=(_B,),
        in_specs=[
            pl.BlockSpec((1, 2048, 128), lambda i: (i, 0, 0)),
            pl.BlockSpec((1, 2048, 128), lambda i: (i, 0, 0)),
        ],
        out_specs=[scalar_spec] * 4,
        out_shape=out_shapes,
    )(pred2, target2)

    hard_focal = jnp.sum(topk) / jnp.float32(_B * _K)
    dice = (2.0 * inter + 1.0) / (ssig + st + 1.0)
    dice_loss = jnp.mean(1.0 - dice)
    return _DICE_WEIGHT * dice_loss + _FOCAL_WEIGHT * hard_focal
